# Initial kernel scaffold; baseline (speedup 1.0000x reference)
import jax
import jax.numpy as jnp
from jax.experimental import pallas as pl


def kernel(scores, segments, gt):
    raise NotImplementedError("write your pallas kernel here")


# speedup vs baseline: 19.0274x; 19.0274x over previous
import functools

import jax
import jax.numpy as jnp
from jax import lax
from jax.experimental import pallas as pl
from jax.experimental.pallas import tpu as pltpu

_N = 5000
_M = 100
_NP = 5120
_ROWS = _NP // 128
_BIG = 1.0e9


def _ap_body(amin_ref, amax_ref, conf_ref, conf_smem, bmin_smem, bmax_smem,
             out_ref):
    amin = amin_ref[...]
    amax = amax_ref[...]
    conf = conf_ref[...]
    alen = amax - amin

    iota = (lax.broadcasted_iota(jnp.int32, (_ROWS, 128), 0) * 128
            + lax.broadcasted_iota(jnp.int32, (_ROWS, 128), 1)
            ).astype(jnp.float32)
    lbl_row = lax.broadcasted_iota(jnp.int32, (1, 128), 1).astype(jnp.float32)
    lbl_col = lax.broadcasted_iota(jnp.int32, (128, 1), 0).astype(jnp.float32)

    def step(j, carry):
        tp, rank_row, rank_col = carry
        b0 = bmin_smem[j]
        b1 = bmax_smem[j]
        inter = jnp.maximum(jnp.minimum(amax, b1) - jnp.maximum(amin, b0), 0.0)
        union = alen + (b1 - b0) - inter
        iou = inter / union
        cand = (iou > 0.5) & (tp == 0.0)
        m = jnp.min(jnp.where(cand, iota, _BIG))
        has = m < _BIG
        tp = jnp.where((iota == m) & has, 1.0, tp)
        mi = jnp.minimum(m, float(_NP - 1)).astype(jnp.int32)
        c = conf_smem[mi]
        cnt = jnp.where(conf > c, 1.0, 0.0) + jnp.where(
            (conf == c) & (iota < m), 1.0, 0.0)
        rank = jnp.sum(cnt)
        rank = jnp.where(has, rank, -5.0)
        jf = j.astype(jnp.float32)
        rank_row = jnp.where(lbl_row == jf, rank, rank_row)
        rank_col = jnp.where(lbl_col == jf, rank, rank_col)
        return tp, rank_row, rank_col

    tp0 = jnp.zeros((_ROWS, 128), jnp.float32)
    rr0 = jnp.full((1, 128), -5.0, jnp.float32)
    rc0 = jnp.full((128, 1), -5.0, jnp.float32)
    _, rank_row, rank_col = lax.fori_loop(0, _M, step, (tp0, rr0, rc0))

    valid_row = rank_row >= 0.0
    valid_col = rank_col >= 0.0

    le = valid_col & (rank_col <= rank_row)
    acc_row = jnp.sum(jnp.where(le, 1.0, 0.0), axis=0, keepdims=True)
    le2 = valid_row & (rank_row <= rank_col)
    acc_col = jnp.sum(jnp.where(le2, 1.0, 0.0), axis=1, keepdims=True)

    p_col = acc_col / (rank_col + 1.0)
    ge = valid_col & (rank_col >= rank_row)
    suff_row = jnp.max(jnp.where(ge, p_col, 0.0), axis=0, keepdims=True)

    contrib = jnp.where(valid_row & (rank_row >= 1.0), suff_row, 0.0)
    out_ref[0, 0] = jnp.sum(contrib) / float(_M)


@jax.jit
def kernel(scores, segments, gt):
    pad = _NP - _N
    amin = jnp.pad(segments[:, 0], (0, pad), constant_values=-1.0e6)
    amax = jnp.pad(segments[:, 1], (0, pad), constant_values=-1.0e6)
    conf = jnp.pad(scores, (0, pad), constant_values=-1.0)
    amin2 = amin.reshape(_ROWS, 128)
    amax2 = amax.reshape(_ROWS, 128)
    conf2 = conf.reshape(_ROWS, 128)

    out = pl.pallas_call(
        _ap_body,
        out_shape=jax.ShapeDtypeStruct((1, 1), jnp.float32),
        in_specs=[
            pl.BlockSpec(memory_space=pltpu.VMEM),
            pl.BlockSpec(memory_space=pltpu.VMEM),
            pl.BlockSpec(memory_space=pltpu.VMEM),
            pl.BlockSpec(memory_space=pltpu.SMEM),
            pl.BlockSpec(memory_space=pltpu.SMEM),
            pl.BlockSpec(memory_space=pltpu.SMEM),
        ],
        out_specs=pl.BlockSpec(memory_space=pltpu.SMEM),
    )(amin2, amax2, conf2, conf, gt[:, 0], gt[:, 1])
    return out[0, 0]


# speedup vs baseline: 28.1456x; 1.4792x over previous
import jax
import jax.numpy as jnp
from jax import lax
from jax.experimental import pallas as pl
from jax.experimental.pallas import tpu as pltpu

_N = 5000
_M = 100
_NP = 5120
_ROWS = _NP // 128
_BIG = 1.0e9


def _ap_body(amin_ref, amax_ref, conf_ref, bmin_ref, bmax_ref, out_ref,
             ptp_ref):
    amin = amin_ref[...]
    amax = amax_ref[...]
    conf = conf_ref[...]
    alen = amax - amin

    iota = (lax.broadcasted_iota(jnp.int32, (_ROWS, 128), 0) * 128
            + lax.broadcasted_iota(jnp.int32, (_ROWS, 128), 1)
            ).astype(jnp.float32)
    lbl_col = lax.broadcasted_iota(jnp.int32, (128, 1), 0).astype(jnp.float32)

    bmin3 = bmin_ref[...].reshape(128, 1, 1)
    bmax3 = bmax_ref[...].reshape(128, 1, 1)
    amin3 = amin.reshape(1, _ROWS, 128)
    amax3 = amax.reshape(1, _ROWS, 128)
    inter = jnp.maximum(
        jnp.minimum(amax3, bmax3) - jnp.maximum(amin3, bmin3), 0.0)
    union = (amax3 - amin3) + (bmax3 - bmin3) - inter
    iou = inter / union
    ptp_ref[...] = jnp.where(iou > 0.5, 1.0, 0.0)

    def step(j, carry):
        tp, chosen_col = carry
        row = ptp_ref[j]
        cand = (row > 0.0) & (tp == 0.0)
        m11 = jnp.min(jnp.where(cand, iota, _BIG), axis=(0, 1), keepdims=True)
        mb = jnp.broadcast_to(m11, (_ROWS, 128))
        tp = jnp.where((iota == mb) & (mb < _BIG), 1.0, tp)
        jf = j.astype(jnp.float32)
        chosen_col = jnp.where(lbl_col == jf,
                               jnp.broadcast_to(m11, (128, 1)), chosen_col)
        return tp, chosen_col

    tp0 = jnp.zeros((_ROWS, 128), jnp.float32)
    ch0 = jnp.full((128, 1), _BIG, jnp.float32)
    _, chosen_col = lax.fori_loop(0, _M, step, (tp0, ch0))

    valid_col = chosen_col < _BIG
    validf_col = jnp.where(valid_col, 1.0, 0.0)

    chosen3 = chosen_col.reshape(128, 1, 1)
    iota3 = iota.reshape(1, _ROWS, 128)
    conf3 = conf.reshape(1, _ROWS, 128)
    onehot = jnp.where(iota3 == chosen3, 1.0, 0.0)
    cc_col = jnp.sum(onehot * conf3, axis=(1, 2), keepdims=True)
    cnt = (jnp.where(conf3 > cc_col, 1.0, 0.0)
           + jnp.where((conf3 == cc_col) & (iota3 < chosen3), 1.0, 0.0))
    rank3 = jnp.sum(cnt, axis=(1, 2), keepdims=True)
    rank_col = jnp.where(valid_col, rank3.reshape(128, 1), -5.0)

    eye = jnp.where(
        lax.broadcasted_iota(jnp.int32, (128, 128), 0)
        == lax.broadcasted_iota(jnp.int32, (128, 128), 1), 1.0, 0.0)
    dnums = (((0,), (0,)), ((), ()))
    rank_row = lax.dot_general(rank_col, eye, dnums,
                               preferred_element_type=jnp.float32)
    validf_row = lax.dot_general(validf_col, eye, dnums,
                                 preferred_element_type=jnp.float32)

    le = (rank_col <= rank_row) & (validf_col > 0.0)
    acc_row = jnp.sum(jnp.where(le, 1.0, 0.0), axis=0, keepdims=True)
    le2 = (rank_row <= rank_col) & (validf_row > 0.0)
    acc_col = jnp.sum(jnp.where(le2, 1.0, 0.0), axis=1, keepdims=True)
    p_col = acc_col / (rank_col + 1.0)
    ge = (rank_col >= rank_row) & (validf_col > 0.0)
    suff_row = jnp.max(jnp.where(ge, p_col, 0.0), axis=0, keepdims=True)
    contrib = jnp.where((validf_row > 0.0) & (rank_row >= 1.0), suff_row, 0.0)
    out_ref[0, 0] = jnp.sum(contrib) / float(_M)


@jax.jit
def kernel(scores, segments, gt):
    pad = _NP - _N
    amin = jnp.pad(segments[:, 0], (0, pad), constant_values=-1.0e6)
    amax = jnp.pad(segments[:, 1], (0, pad), constant_values=-1.0e6)
    conf = jnp.pad(scores, (0, pad), constant_values=-1.0)
    bmin = jnp.pad(gt[:, 0], (0, 28), constant_values=2.0e6)
    bmax = jnp.pad(gt[:, 1], (0, 28), constant_values=2.0e6)

    out = pl.pallas_call(
        _ap_body,
        out_shape=jax.ShapeDtypeStruct((1, 1), jnp.float32),
        in_specs=[
            pl.BlockSpec(memory_space=pltpu.VMEM),
            pl.BlockSpec(memory_space=pltpu.VMEM),
            pl.BlockSpec(memory_space=pltpu.VMEM),
            pl.BlockSpec(memory_space=pltpu.VMEM),
            pl.BlockSpec(memory_space=pltpu.VMEM),
        ],
        out_specs=pl.BlockSpec(memory_space=pltpu.SMEM),
        scratch_shapes=[pltpu.VMEM((128, _ROWS, 128), jnp.float32)],
    )(amin.reshape(_ROWS, 128), amax.reshape(_ROWS, 128),
      conf.reshape(_ROWS, 128), bmin.reshape(128, 1), bmax.reshape(128, 1))
    return out[0, 0]


# speedup vs baseline: 28.2486x; 1.0037x over previous
import jax
import jax.numpy as jnp
from jax import lax
from jax.experimental import pallas as pl
from jax.experimental.pallas import tpu as pltpu

_N = 5000
_M = 100
_NP = 5120
_ROWS = _NP // 128
_BIG = 1.0e9


def _ap_body(amin_ref, amax_ref, conf_ref, bmin_ref, bmax_ref, out_ref,
             ptp_ref):
    amin = amin_ref[...]
    amax = amax_ref[...]
    conf = conf_ref[...]
    alen = amax - amin

    iota = (lax.broadcasted_iota(jnp.int32, (_ROWS, 128), 0) * 128
            + lax.broadcasted_iota(jnp.int32, (_ROWS, 128), 1)
            ).astype(jnp.float32)
    lbl_col = lax.broadcasted_iota(jnp.int32, (128, 1), 0).astype(jnp.float32)

    bmin3 = bmin_ref[...].reshape(128, 1, 1)
    bmax3 = bmax_ref[...].reshape(128, 1, 1)
    amin3 = amin.reshape(1, _ROWS, 128)
    amax3 = amax.reshape(1, _ROWS, 128)
    inter = jnp.maximum(
        jnp.minimum(amax3, bmax3) - jnp.maximum(amin3, bmin3), 0.0)
    union = (amax3 - amin3) + (bmax3 - bmin3) - inter
    iou = inter / union
    ptp_ref[...] = jnp.where(iou > 0.5, 1.0, 0.0)

    def step(j, carry):
        tp, chosen_col = carry
        row = ptp_ref[j]
        cand = (row > 0.0) & (tp == 0.0)
        m11 = jnp.min(jnp.where(cand, iota, _BIG), axis=(0, 1), keepdims=True)
        mb = jnp.broadcast_to(m11, (_ROWS, 128))
        tp = jnp.where((iota == mb) & (mb < _BIG), 1.0, tp)
        jf = j.astype(jnp.float32)
        chosen_col = jnp.where(lbl_col == jf,
                               jnp.broadcast_to(m11, (128, 1)), chosen_col)
        return tp, chosen_col

    tp0 = jnp.zeros((_ROWS, 128), jnp.float32)
    ch0 = jnp.full((128, 1), _BIG, jnp.float32)
    _, chosen_col = lax.fori_loop(0, _M, step, (tp0, ch0))

    valid_col = chosen_col < _BIG
    validf_col = jnp.where(valid_col, 1.0, 0.0)

    chosen3 = chosen_col.reshape(128, 1, 1)
    iota3 = iota.reshape(1, _ROWS, 128)
    conf3 = conf.reshape(1, _ROWS, 128)
    onehot = jnp.where(iota3 == chosen3, 1.0, 0.0)
    cc_col = jnp.sum(onehot * conf3, axis=(1, 2), keepdims=True)
    cnt = (jnp.where(conf3 > cc_col, 1.0, 0.0)
           + jnp.where((conf3 == cc_col) & (iota3 < chosen3), 1.0, 0.0))
    rank3 = jnp.sum(cnt, axis=(1, 2), keepdims=True)
    rank_col = jnp.where(valid_col, rank3.reshape(128, 1), -5.0)

    eye = (lax.broadcasted_iota(jnp.int32, (128, 128), 0)
           == lax.broadcasted_iota(jnp.int32, (128, 128), 1))
    rank_row = jnp.sum(jnp.where(eye, rank_col, 0.0), axis=0, keepdims=True)
    validf_row = jnp.sum(jnp.where(eye, validf_col, 0.0), axis=0,
                         keepdims=True)

    le = (rank_col <= rank_row) & (validf_col > 0.0)
    acc_row = jnp.sum(jnp.where(le, 1.0, 0.0), axis=0, keepdims=True)
    le2 = (rank_row <= rank_col) & (validf_row > 0.0)
    acc_col = jnp.sum(jnp.where(le2, 1.0, 0.0), axis=1, keepdims=True)
    p_col = acc_col / (rank_col + 1.0)
    ge = (rank_col >= rank_row) & (validf_col > 0.0)
    suff_row = jnp.max(jnp.where(ge, p_col, 0.0), axis=0, keepdims=True)
    contrib = jnp.where((validf_row > 0.0) & (rank_row >= 1.0), suff_row, 0.0)
    out_ref[0, 0] = jnp.sum(contrib) / float(_M)


@jax.jit
def kernel(scores, segments, gt):
    pad = _NP - _N
    amin = jnp.pad(segments[:, 0], (0, pad), constant_values=-1.0e6)
    amax = jnp.pad(segments[:, 1], (0, pad), constant_values=-1.0e6)
    conf = jnp.pad(scores, (0, pad), constant_values=-1.0)
    bmin = jnp.pad(gt[:, 0], (0, 28), constant_values=2.0e6)
    bmax = jnp.pad(gt[:, 1], (0, 28), constant_values=2.0e6)

    out = pl.pallas_call(
        _ap_body,
        out_shape=jax.ShapeDtypeStruct((1, 1), jnp.float32),
        in_specs=[
            pl.BlockSpec(memory_space=pltpu.VMEM),
            pl.BlockSpec(memory_space=pltpu.VMEM),
            pl.BlockSpec(memory_space=pltpu.VMEM),
            pl.BlockSpec(memory_space=pltpu.VMEM),
            pl.BlockSpec(memory_space=pltpu.VMEM),
        ],
        out_specs=pl.BlockSpec(memory_space=pltpu.SMEM),
        scratch_shapes=[pltpu.VMEM((128, _ROWS, 128), jnp.float32)],
    )(amin.reshape(_ROWS, 128), amax.reshape(_ROWS, 128),
      conf.reshape(_ROWS, 128), bmin.reshape(128, 1), bmax.reshape(128, 1))
    return out[0, 0]
